# baseline (speedup 1.0000x reference)
import functools
import math

import jax
import jax.numpy as jnp
from jax import lax
from jax.experimental import pallas as pl
from jax.experimental.pallas import tpu as pltpu

NDIM = 3
ATOM_NF = 16
RES_NF = 21
JOINT = 16
HID = 64
NB = 16
NORM_FACTOR = 100.0
PAD_COORD = 8
F32 = jnp.float32


def _silu(x):
    return x * jax.nn.sigmoid(x)


HIGH = lax.Precision.HIGHEST


def _dot(a, b):
    return jnp.dot(a, b, preferred_element_type=F32)


def _pdot(a, b):
    return lax.dot_general(a, b, (((1,), (1,)), ((), ())),
                           precision=HIGH, preferred_element_type=F32)


def _coord_rows(xj):
    eye = (lax.broadcasted_iota(jnp.int32, (NDIM, PAD_COORD), 0) ==
           lax.broadcasted_iota(jnp.int32, (NDIM, PAD_COORD), 1)).astype(F32)
    return _pdot(eye, xj)


def _diff_planes(xi, xjr):
    return [xi[:, k:k + 1] - xjr[k:k + 1, :] for k in range(NDIM)]


def _r2(planes):
    return planes[0] * planes[0] + planes[1] * planes[1] \
        + planes[2] * planes[2]


def _row_block(ref, r0, b):
    return ref[pl.ds(r0, b), :]


def _col_range(m_ref, mi, bjc, seg):
    mcol = m_ref[:, 0:1]
    n = mcol.shape[0]
    idx = lax.broadcasted_iota(jnp.int32, (n, 1), 0)
    lt = mcol < jnp.min(mi)
    le = mcol <= jnp.max(mi)
    s1 = idx < seg
    js1 = jnp.sum((lt & s1).astype(jnp.int32))
    je1 = jnp.sum((le & s1).astype(jnp.int32))
    js2 = seg + jnp.sum((lt & (~s1)).astype(jnp.int32))
    je2 = seg + jnp.sum((le & (~s1)).astype(jnp.int32))
    a0 = js1 // bjc
    a1 = (je1 + bjc - 1) // bjc
    b0 = js2 // bjc
    b1 = (je2 + bjc - 1) // bjc
    return a0, a1, jnp.maximum(b0, a1), b1


def _gcl_kernel(h_ref, x_ref, x0_ref, m_ref,
                eW1, eb1, eW2, eb2, nW1, nb1, nW2, nb2,
                out, acc, *, bi, bjc, cutoff, seg):
    r0 = pl.program_id(0) * bi
    hi = _row_block(h_ref, r0, bi)
    xi = _row_block(x_ref, r0, bi)
    x0i = _row_block(x0_ref, r0, bi)
    mi = m_ref[pl.ds(r0, bi), 0:1]
    a0, a1, b0, b1 = _col_range(m_ref, mi, bjc, seg)
    acc[...] = jnp.zeros_like(acc)

    def body(jb, carry):
        c0 = jb * bjc
        hj = _row_block(h_ref, c0, bjc)
        r = _r2(_diff_planes(xi, _coord_rows(_row_block(x_ref, c0, bjc))))
        d0 = _r2(_diff_planes(x0i,
                              _coord_rows(_row_block(x0_ref, c0, bjc))))
        mj_row = _pdot(jnp.ones((1, PAD_COORD), F32),
                       _row_block(m_ref, c0, bjc))
        w = (mi == mj_row).astype(F32)
        if cutoff:
            w = w * (d0 <= 9.0).astype(F32)
        inp = jnp.concatenate(
            [jnp.broadcast_to(hi[:, None, :], (bi, bjc, HID)),
             jnp.broadcast_to(hj[None, :, :], (bi, bjc, HID)),
             r[:, :, None], d0[:, :, None]],
            axis=-1).reshape(bi * bjc, 2 * HID + 2)
        t1 = _silu(_dot(inp, eW1[...]) + eb1[...])
        M = _silu(_dot(t1, eW2[...])
                  + eb2[...]).reshape(bi, bjc, HID)
        acc[...] += jnp.sum(M * w[:, :, None], axis=1)
        return carry

    lax.fori_loop(a0, a1, body, 0)
    lax.fori_loop(b0, b1, body, 0)
    agg = acc[...] * (1.0 / NORM_FACTOR)
    z = jnp.concatenate([hi, agg], axis=1)
    t = _silu(_dot(z, nW1[...]) + nb1[...])
    out[...] = hi + _dot(t, nW2[...]) + nb2[...]


def _coord_kernel(h_ref, x_ref, x0_ref, m_ref,
                  cW1, cb1, cW2, cb2, cW3,
                  out, acc_v, *, bi, bjc, cutoff, seg):
    r0 = pl.program_id(0) * bi
    hi = _row_block(h_ref, r0, bi)
    xi = _row_block(x_ref, r0, bi)
    x0i = _row_block(x0_ref, r0, bi)
    mi = m_ref[pl.ds(r0, bi), 0:1]
    a0, a1, b0, b1 = _col_range(m_ref, mi, bjc, seg)
    acc_v[...] = jnp.zeros_like(acc_v)

    def body(jb, carry):
        c0 = jb * bjc
        hj = _row_block(h_ref, c0, bjc)
        planes = _diff_planes(xi, _coord_rows(_row_block(x_ref, c0, bjc)))
        r = _r2(planes)
        d0 = _r2(_diff_planes(x0i,
                              _coord_rows(_row_block(x0_ref, c0, bjc))))
        mj_row = _pdot(jnp.ones((1, PAD_COORD), F32),
                       _row_block(m_ref, c0, bjc))
        w = (mi == mj_row).astype(F32)
        if cutoff:
            w = w * (d0 <= 9.0).astype(F32)
        inp = jnp.concatenate(
            [jnp.broadcast_to(hi[:, None, :], (bi, bjc, HID)),
             jnp.broadcast_to(hj[None, :, :], (bi, bjc, HID)),
             r[:, :, None], d0[:, :, None]],
            axis=-1).reshape(bi * bjc, 2 * HID + 2)
        t1 = _silu(_dot(inp, cW1[...]) + cb1[...])
        t2 = _silu(_dot(t1, cW2[...])
                   + cb2[...])
        phi = _dot(t2, cW3[...]).reshape(bi, bjc)
        c = phi * w / jnp.sqrt(r + 1e-8)
        for k in range(NDIM):
            acc_v[:, k:k + 1] += jnp.sum(planes[k] * c, axis=1, keepdims=True)
        return carry

    lax.fori_loop(a0, a1, body, 0)
    lax.fori_loop(b0, b1, body, 0)
    out[...] = xi + acc_v[...] * (1.0 / NORM_FACTOR)


def _edge_pass(kind, h, x, x0, m, weights, *, cutoff, seg, bi=16, bjc=128):
    n = h.shape[0]
    ni = n // bi
    full = lambda a: pl.BlockSpec(a.shape, lambda i: (0,) * a.ndim)
    in_specs = [full(h), full(x), full(x0), full(m)] + [full(w)
                                                        for w in weights]
    if kind == 'gcl':
        body = functools.partial(_gcl_kernel, bi=bi, bjc=bjc, cutoff=cutoff,
                                 seg=seg)
        out_shape = jax.ShapeDtypeStruct((n, HID), F32)
        out_spec = pl.BlockSpec((bi, HID), lambda i: (i, 0))
        scratch = [pltpu.VMEM((bi, HID), F32)]
    else:
        body = functools.partial(_coord_kernel, bi=bi, bjc=bjc, cutoff=cutoff,
                                 seg=seg)
        out_shape = jax.ShapeDtypeStruct((n, PAD_COORD), F32)
        out_spec = pl.BlockSpec((bi, PAD_COORD), lambda i: (i, 0))
        scratch = [pltpu.VMEM((bi, PAD_COORD), F32)]
    return pl.pallas_call(
        body,
        grid=(ni,),
        in_specs=in_specs,
        out_specs=out_spec,
        out_shape=out_shape,
        scratch_shapes=scratch,
        compiler_params=pltpu.CompilerParams(
            dimension_semantics=("arbitrary",)),
    )(h, x, x0, m, *weights)


def _mlp2_kernel(x, W1, b1, W2, b2, o):
    t = _silu(_dot(x[...], W1[...]) + b1[...])
    o[...] = _dot(t, W2[...]) + b2[...]


def _mlp2(x, lp):
    (W1, b1), (W2, b2) = lp
    return pl.pallas_call(
        _mlp2_kernel,
        out_shape=jax.ShapeDtypeStruct((x.shape[0], W2.shape[1]), F32),
    )(x, W1, b1[None, :], W2, b2[None, :])


def _linear_kernel(x, W, b, o):
    o[...] = _dot(x[...], W[...]) + b[...]


def _linear(x, W, b):
    return pl.pallas_call(
        _linear_kernel,
        out_shape=jax.ShapeDtypeStruct((x.shape[0], W.shape[1]), F32),
    )(x, W, b[None, :])


def _vel_center_kernel(xf, x0, m, o):
    vel = xf[...] - x0[...]
    ids = lax.broadcasted_iota(jnp.int32, (1, NB), 1).astype(F32)
    onehot = (m[:, 0:1] == ids).astype(F32)
    s = lax.dot_general(onehot, vel, (((0,), (0,)), ((), ())),
                        precision=HIGH, preferred_element_type=F32)
    cnt = lax.dot_general(onehot, jnp.ones_like(vel[:, 0:1]),
                          (((0,), (0,)), ((), ())),
                          precision=HIGH, preferred_element_type=F32)
    mean = s / jnp.maximum(cnt, 1.0)
    o[...] = vel - _dot(onehot, mean)


def _vel_center(x_final, x_init, m):
    return pl.pallas_call(
        _vel_center_kernel,
        out_shape=jax.ShapeDtypeStruct(x_final.shape, F32),
    )(x_final, x_init, m)


def _pad_nodes(x, h, mask, n_pad):
    n = x.shape[0]
    xp = jnp.zeros((n_pad, PAD_COORD), F32).at[:n, :NDIM].set(x)
    hp = jnp.zeros((n_pad, HID), F32).at[:n].set(h)
    mcol = jnp.full((n_pad, 1), 255.0, F32).at[:n, 0].set(mask.astype(F32))
    mp = jnp.concatenate([mcol, jnp.zeros((n_pad, PAD_COORD - 1), F32)], axis=1)
    return xp, hp, mp


def kernel(xh_atoms, xh_residues, xh_full, t, mask_atoms, mask_residues,
           mask_full, params):
    na = xh_atoms.shape[0]
    nr = xh_residues.shape[0]
    nf = xh_full.shape[0]
    n1 = na + nr
    n2 = nr + nf
    B = 128
    n1p = -(-n1 // B) * B
    n2p = -(-n2 // B) * B

    x_a = xh_atoms[:, :NDIM]
    x_r = xh_residues[:, :NDIM]
    x_f = xh_full[:, :NDIM]
    h_a = _mlp2(xh_atoms[:, NDIM:], params['atom_enc'])
    h_r = _mlp2(xh_residues[:, NDIM:], params['res_enc'])
    h_f = _mlp2(xh_full[:, NDIM:], params['res_enc'])

    tval = t.reshape(())
    We, be = params['emb']

    def embed(hj):
        h17 = jnp.concatenate(
            [hj, jnp.full((hj.shape[0], 1), 1.0, F32) * tval], axis=1)
        return _linear(h17, We, be)

    h1 = embed(jnp.concatenate([h_a, h_r], axis=0))
    h2 = embed(jnp.concatenate([h_r, h_f], axis=0))
    x1 = jnp.concatenate([x_a, x_r], axis=0)
    x2 = jnp.concatenate([x_r, x_f], axis=0)
    m1 = jnp.concatenate([mask_atoms, mask_residues])
    m2 = jnp.concatenate([mask_residues, mask_full])

    x1p, h1p, m1p = _pad_nodes(x1, h1, m1, n1p)
    x2p, h2p, m2p = _pad_nodes(x2, h2, m2, n2p)
    x01 = x1p
    x02 = x2p

    def edge_w(g, which):
        if which == 'coord':
            (W1, b1), (W2, b2), (W3, _) = g['coord']
            return (W1, b1[None, :], W2, b2[None, :], W3)
        (W1, b1), (W2, b2) = which['edge']
        (Wn1, bn1), (Wn2, bn2) = which['node']
        return (W1, b1[None, :], W2, b2[None, :],
                Wn1, bn1[None, :], Wn2, bn2[None, :])

    stacked = jax.tree.map(lambda *a: jnp.stack(a), *params['layers'])

    def layer(carry, lw):
        h1, x1, h2, x2 = carry
        for g in lw['gcls']:
            h1 = _edge_pass('gcl', h1, x1, x01, m1p, edge_w(lw, g),
                            cutoff=False, seg=na)
        x1n = _edge_pass('coord', h1, x1, x01, m1p, edge_w(lw, 'coord'),
                         cutoff=False, seg=na)
        for g in lw['gcls']:
            h2 = _edge_pass('gcl', h2, x2, x02, m2p, edge_w(lw, g),
                            cutoff=True, seg=nr)
        x2n = _edge_pass('coord', h2, x2, x02, m2p, edge_w(lw, 'coord'),
                         cutoff=True, seg=nr)
        x1, x2 = x1n, x2n
        hr = 0.5 * (h1[n1 - nr:n1] + h2[:nr])
        xr = 0.5 * (x1[n1 - nr:n1] + x2[:nr])
        h1 = jnp.concatenate([h1[:n1 - nr], hr, h1[n1:]], axis=0)
        x1 = jnp.concatenate([x1[:n1 - nr], xr, x1[n1:]], axis=0)
        h2 = jnp.concatenate([hr, h2[nr:]], axis=0)
        x2 = jnp.concatenate([xr, x2[nr:]], axis=0)
        return (h1, x1, h2, x2), None

    (h1p, x1p, h2p, x2p), _ = lax.scan(layer, (h1p, x1p, h2p, x2p), stacked)

    Wo, bo = params['emb_out']
    h_final = _linear(h1p[:n1], Wo, bo)[:, :JOINT]
    h_fa = _mlp2(h_final[:na], params['atom_dec'])
    h_fr = _mlp2(h_final[na:], params['res_dec'])

    vel = _vel_center(x1p, x01, m1p)[:n1, :NDIM]
    return (jnp.concatenate([vel[:na], h_fa], axis=-1),
            jnp.concatenate([vel[na:], h_fr], axis=-1))


# speedup vs baseline: 1.3145x; 1.3145x over previous
import functools
import math

import jax
import jax.numpy as jnp
from jax import lax
from jax.experimental import pallas as pl
from jax.experimental.pallas import tpu as pltpu

NDIM = 3
ATOM_NF = 16
RES_NF = 21
JOINT = 16
HID = 64
NB = 16
NORM_FACTOR = 100.0
PAD_COORD = 8
F32 = jnp.float32


def _silu(x):
    return x * jax.nn.sigmoid(x)


HIGH = lax.Precision.HIGHEST


def _dot(a, b):
    return jnp.dot(a, b, preferred_element_type=F32)


def _pdot(a, b):
    return lax.dot_general(a, b, (((1,), (1,)), ((), ())),
                           precision=HIGH, preferred_element_type=F32)


def _coord_rows(xj):
    eye = (lax.broadcasted_iota(jnp.int32, (NDIM, PAD_COORD), 0) ==
           lax.broadcasted_iota(jnp.int32, (NDIM, PAD_COORD), 1)).astype(F32)
    return _pdot(eye, xj)


def _diff_planes(xi, xjr):
    return [xi[:, k:k + 1] - xjr[k:k + 1, :] for k in range(NDIM)]


def _r2(planes):
    return planes[0] * planes[0] + planes[1] * planes[1] \
        + planes[2] * planes[2]


def _row_block(ref, r0, b):
    return ref[pl.ds(r0, b), :]


def _col_range(m_ref, mi, bjc, seg):
    mcol = m_ref[:, 0:1]
    n = mcol.shape[0]
    idx = lax.broadcasted_iota(jnp.int32, (n, 1), 0)
    lt = mcol < jnp.min(mi)
    le = mcol <= jnp.max(mi)
    s1 = idx < seg
    js1 = jnp.sum((lt & s1).astype(jnp.int32))
    je1 = jnp.sum((le & s1).astype(jnp.int32))
    js2 = seg + jnp.sum((lt & (~s1)).astype(jnp.int32))
    je2 = seg + jnp.sum((le & (~s1)).astype(jnp.int32))
    a0 = js1 // bjc
    a1 = (je1 + bjc - 1) // bjc
    b0 = js2 // bjc
    b1 = (je2 + bjc - 1) // bjc
    return a0, a1, jnp.maximum(b0, a1), b1


def _gcl_kernel(h_ref, x_ref, x0_ref, m_ref,
                eW1, eb1, eW2, eb2, nW1, nb1, nW2, nb2,
                out, acc, *, bi, bjc, cutoff, seg):
    r0 = pl.program_id(0) * bi
    hi = _row_block(h_ref, r0, bi)
    xi = _row_block(x_ref, r0, bi)
    x0i = _row_block(x0_ref, r0, bi)
    mi = m_ref[pl.ds(r0, bi), 0:1]
    a0, a1, b0, b1 = _col_range(m_ref, mi, bjc, seg)
    acc[...] = jnp.zeros_like(acc)

    def body(jb, carry):
        c0 = jb * bjc
        hj = _row_block(h_ref, c0, bjc)
        r = _r2(_diff_planes(xi, _coord_rows(_row_block(x_ref, c0, bjc))))
        d0 = _r2(_diff_planes(x0i,
                              _coord_rows(_row_block(x0_ref, c0, bjc))))
        mj_row = _pdot(jnp.ones((1, PAD_COORD), F32),
                       _row_block(m_ref, c0, bjc))
        w = (mi == mj_row).astype(F32)
        if cutoff:
            w = w * (d0 <= 9.0).astype(F32)
        inp = jnp.concatenate(
            [jnp.broadcast_to(hi[:, None, :], (bi, bjc, HID)),
             jnp.broadcast_to(hj[None, :, :], (bi, bjc, HID)),
             r[:, :, None], d0[:, :, None]],
            axis=-1).reshape(bi * bjc, 2 * HID + 2)
        t1 = _silu(_dot(inp, eW1[...]) + eb1[...])
        M = _silu(_dot(t1, eW2[...])
                  + eb2[...]).reshape(bi, bjc, HID)
        acc[...] += jnp.sum(M * w[:, :, None], axis=1)
        return carry

    lax.fori_loop(a0, a1, body, 0)
    lax.fori_loop(b0, b1, body, 0)
    agg = acc[...] * (1.0 / NORM_FACTOR)
    z = jnp.concatenate([hi, agg], axis=1)
    t = _silu(_dot(z, nW1[...]) + nb1[...])
    out[...] = hi + _dot(t, nW2[...]) + nb2[...]


def _coord_kernel(h_ref, x_ref, x0_ref, m_ref,
                  cW1, cb1, cW2, cb2, cW3,
                  out, acc_v, *, bi, bjc, cutoff, seg):
    r0 = pl.program_id(0) * bi
    hi = _row_block(h_ref, r0, bi)
    xi = _row_block(x_ref, r0, bi)
    x0i = _row_block(x0_ref, r0, bi)
    mi = m_ref[pl.ds(r0, bi), 0:1]
    a0, a1, b0, b1 = _col_range(m_ref, mi, bjc, seg)
    acc_v[...] = jnp.zeros_like(acc_v)

    def body(jb, carry):
        c0 = jb * bjc
        hj = _row_block(h_ref, c0, bjc)
        planes = _diff_planes(xi, _coord_rows(_row_block(x_ref, c0, bjc)))
        r = _r2(planes)
        d0 = _r2(_diff_planes(x0i,
                              _coord_rows(_row_block(x0_ref, c0, bjc))))
        mj_row = _pdot(jnp.ones((1, PAD_COORD), F32),
                       _row_block(m_ref, c0, bjc))
        w = (mi == mj_row).astype(F32)
        if cutoff:
            w = w * (d0 <= 9.0).astype(F32)
        inp = jnp.concatenate(
            [jnp.broadcast_to(hi[:, None, :], (bi, bjc, HID)),
             jnp.broadcast_to(hj[None, :, :], (bi, bjc, HID)),
             r[:, :, None], d0[:, :, None]],
            axis=-1).reshape(bi * bjc, 2 * HID + 2)
        t1 = _silu(_dot(inp, cW1[...]) + cb1[...])
        t2 = _silu(_dot(t1, cW2[...])
                   + cb2[...])
        phi = _dot(t2, cW3[...]).reshape(bi, bjc)
        c = phi * w / jnp.sqrt(r + 1e-8)
        for k in range(NDIM):
            acc_v[:, k:k + 1] += jnp.sum(planes[k] * c, axis=1, keepdims=True)
        return carry

    lax.fori_loop(a0, a1, body, 0)
    lax.fori_loop(b0, b1, body, 0)
    out[...] = xi + acc_v[...] * (1.0 / NORM_FACTOR)


def _edge_pass(kind, h, x, x0, m, weights, *, cutoff, seg, bi=32, bjc=64):
    n = h.shape[0]
    ni = n // bi
    full = lambda a: pl.BlockSpec(a.shape, lambda i: (0,) * a.ndim)
    in_specs = [full(h), full(x), full(x0), full(m)] + [full(w)
                                                        for w in weights]
    if kind == 'gcl':
        body = functools.partial(_gcl_kernel, bi=bi, bjc=bjc, cutoff=cutoff,
                                 seg=seg)
        out_shape = jax.ShapeDtypeStruct((n, HID), F32)
        out_spec = pl.BlockSpec((bi, HID), lambda i: (i, 0))
        scratch = [pltpu.VMEM((bi, HID), F32)]
    else:
        body = functools.partial(_coord_kernel, bi=bi, bjc=bjc, cutoff=cutoff,
                                 seg=seg)
        out_shape = jax.ShapeDtypeStruct((n, PAD_COORD), F32)
        out_spec = pl.BlockSpec((bi, PAD_COORD), lambda i: (i, 0))
        scratch = [pltpu.VMEM((bi, PAD_COORD), F32)]
    return pl.pallas_call(
        body,
        grid=(ni,),
        in_specs=in_specs,
        out_specs=out_spec,
        out_shape=out_shape,
        scratch_shapes=scratch,
        compiler_params=pltpu.CompilerParams(
            dimension_semantics=("arbitrary",)),
    )(h, x, x0, m, *weights)


def _mlp2_kernel(x, W1, b1, W2, b2, o):
    t = _silu(_dot(x[...], W1[...]) + b1[...])
    o[...] = _dot(t, W2[...]) + b2[...]


def _mlp2(x, lp):
    (W1, b1), (W2, b2) = lp
    return pl.pallas_call(
        _mlp2_kernel,
        out_shape=jax.ShapeDtypeStruct((x.shape[0], W2.shape[1]), F32),
    )(x, W1, b1[None, :], W2, b2[None, :])


def _linear_kernel(x, W, b, o):
    o[...] = _dot(x[...], W[...]) + b[...]


def _linear(x, W, b):
    return pl.pallas_call(
        _linear_kernel,
        out_shape=jax.ShapeDtypeStruct((x.shape[0], W.shape[1]), F32),
    )(x, W, b[None, :])


def _vel_center_kernel(xf, x0, m, o):
    vel = xf[...] - x0[...]
    ids = lax.broadcasted_iota(jnp.int32, (1, NB), 1).astype(F32)
    onehot = (m[:, 0:1] == ids).astype(F32)
    s = lax.dot_general(onehot, vel, (((0,), (0,)), ((), ())),
                        precision=HIGH, preferred_element_type=F32)
    cnt = lax.dot_general(onehot, jnp.ones_like(vel[:, 0:1]),
                          (((0,), (0,)), ((), ())),
                          precision=HIGH, preferred_element_type=F32)
    mean = s / jnp.maximum(cnt, 1.0)
    o[...] = vel - _dot(onehot, mean)


def _vel_center(x_final, x_init, m):
    return pl.pallas_call(
        _vel_center_kernel,
        out_shape=jax.ShapeDtypeStruct(x_final.shape, F32),
    )(x_final, x_init, m)


def _pad_nodes(x, h, mask, n_pad):
    n = x.shape[0]
    xp = jnp.zeros((n_pad, PAD_COORD), F32).at[:n, :NDIM].set(x)
    hp = jnp.zeros((n_pad, HID), F32).at[:n].set(h)
    mcol = jnp.full((n_pad, 1), 255.0, F32).at[:n, 0].set(mask.astype(F32))
    mp = jnp.concatenate([mcol, jnp.zeros((n_pad, PAD_COORD - 1), F32)], axis=1)
    return xp, hp, mp


def kernel(xh_atoms, xh_residues, xh_full, t, mask_atoms, mask_residues,
           mask_full, params):
    na = xh_atoms.shape[0]
    nr = xh_residues.shape[0]
    nf = xh_full.shape[0]
    n1 = na + nr
    n2 = nr + nf
    B = 128
    n1p = -(-n1 // B) * B
    n2p = -(-n2 // B) * B

    x_a = xh_atoms[:, :NDIM]
    x_r = xh_residues[:, :NDIM]
    x_f = xh_full[:, :NDIM]
    h_a = _mlp2(xh_atoms[:, NDIM:], params['atom_enc'])
    h_r = _mlp2(xh_residues[:, NDIM:], params['res_enc'])
    h_f = _mlp2(xh_full[:, NDIM:], params['res_enc'])

    tval = t.reshape(())
    We, be = params['emb']

    def embed(hj):
        h17 = jnp.concatenate(
            [hj, jnp.full((hj.shape[0], 1), 1.0, F32) * tval], axis=1)
        return _linear(h17, We, be)

    h1 = embed(jnp.concatenate([h_a, h_r], axis=0))
    h2 = embed(jnp.concatenate([h_r, h_f], axis=0))
    x1 = jnp.concatenate([x_a, x_r], axis=0)
    x2 = jnp.concatenate([x_r, x_f], axis=0)
    m1 = jnp.concatenate([mask_atoms, mask_residues])
    m2 = jnp.concatenate([mask_residues, mask_full])

    x1p, h1p, m1p = _pad_nodes(x1, h1, m1, n1p)
    x2p, h2p, m2p = _pad_nodes(x2, h2, m2, n2p)
    x01 = x1p
    x02 = x2p

    def edge_w(g, which):
        if which == 'coord':
            (W1, b1), (W2, b2), (W3, _) = g['coord']
            return (W1, b1[None, :], W2, b2[None, :], W3)
        (W1, b1), (W2, b2) = which['edge']
        (Wn1, bn1), (Wn2, bn2) = which['node']
        return (W1, b1[None, :], W2, b2[None, :],
                Wn1, bn1[None, :], Wn2, bn2[None, :])

    stacked = jax.tree.map(lambda *a: jnp.stack(a), *params['layers'])

    def layer(carry, lw):
        h1, x1, h2, x2 = carry
        for g in lw['gcls']:
            h1 = _edge_pass('gcl', h1, x1, x01, m1p, edge_w(lw, g),
                            cutoff=False, seg=na)
        x1n = _edge_pass('coord', h1, x1, x01, m1p, edge_w(lw, 'coord'),
                         cutoff=False, seg=na)
        for g in lw['gcls']:
            h2 = _edge_pass('gcl', h2, x2, x02, m2p, edge_w(lw, g),
                            cutoff=True, seg=nr)
        x2n = _edge_pass('coord', h2, x2, x02, m2p, edge_w(lw, 'coord'),
                         cutoff=True, seg=nr)
        x1, x2 = x1n, x2n
        hr = 0.5 * (h1[n1 - nr:n1] + h2[:nr])
        xr = 0.5 * (x1[n1 - nr:n1] + x2[:nr])
        h1 = jnp.concatenate([h1[:n1 - nr], hr, h1[n1:]], axis=0)
        x1 = jnp.concatenate([x1[:n1 - nr], xr, x1[n1:]], axis=0)
        h2 = jnp.concatenate([hr, h2[nr:]], axis=0)
        x2 = jnp.concatenate([xr, x2[nr:]], axis=0)
        return (h1, x1, h2, x2), None

    (h1p, x1p, h2p, x2p), _ = lax.scan(layer, (h1p, x1p, h2p, x2p), stacked)

    Wo, bo = params['emb_out']
    h_final = _linear(h1p[:n1], Wo, bo)[:, :JOINT]
    h_fa = _mlp2(h_final[:na], params['atom_dec'])
    h_fr = _mlp2(h_final[na:], params['res_dec'])

    vel = _vel_center(x1p, x01, m1p)[:n1, :NDIM]
    return (jnp.concatenate([vel[:na], h_fa], axis=-1),
            jnp.concatenate([vel[na:], h_fr], axis=-1))
